# Initial kernel scaffold; baseline (speedup 1.0000x reference)
import functools

import jax
import jax.numpy as jnp
from jax import lax
from jax.experimental import pallas as pl
from jax.experimental.pallas import tpu as pltpu
from jax.experimental.pallas import tpu_sc as plsc

N = 10000
D = 128
E = 160000
R = 2
L = 16
DA = D + L
NS = 16
N_PAD = 10240
ROWS_PER_SUB = N_PAD // NS
E_PER_SUB = E // NS
CHUNK = 80


def _sc_segment_sum(x_aug, src2, dst2):
    mesh = plsc.VectorSubcoreMesh(core_axis_name="c", subcore_axis_name="s")

    @functools.partial(
        pl.kernel,
        out_type=jax.ShapeDtypeStruct((R, N_PAD, DA), jnp.float32),
        mesh=mesh,
        scratch_types=[
            pltpu.VMEM((CHUNK,), jnp.int32),
            pltpu.VMEM((CHUNK,), jnp.int32),
            pltpu.VMEM((CHUNK, DA), jnp.float32),
            pltpu.VMEM_SHARED((N_PAD, DA), jnp.float32),
        ],
    )
    def sc_kernel(x_hbm, src_hbm, dst_hbm, out_hbm, src_v, dst_v, rows_v, acc_sh):
        c = lax.axis_index("c")
        s = lax.axis_index("s")

        zero = jnp.zeros((L,), jnp.float32)

        @pl.loop(0, CHUNK)
        def _(i):
            @pl.loop(0, DA, step=L)
            def _(j):
                rows_v[i, pl.ds(j, L)] = zero

        row0 = s * ROWS_PER_SUB

        @pl.loop(0, ROWS_PER_SUB, step=CHUNK)
        def _(r0):
            pltpu.sync_copy(rows_v, acc_sh.at[pl.ds(row0 + r0, CHUNK)])

        plsc.subcore_barrier()

        ebase = s * E_PER_SUB

        @pl.loop(0, E_PER_SUB, step=CHUNK)
        def _(e0):
            pltpu.sync_copy(src_hbm.at[c, pl.ds(ebase + e0, CHUNK)], src_v)
            pltpu.sync_copy(dst_hbm.at[c, pl.ds(ebase + e0, CHUNK)], dst_v)
            pltpu.sync_copy(x_hbm.at[src_v], rows_v)
            pltpu.sync_copy(rows_v, acc_sh.at[dst_v], add=True)

        plsc.subcore_barrier()

        pltpu.sync_copy(acc_sh.at[pl.ds(row0, ROWS_PER_SUB)],
                        out_hbm.at[c, pl.ds(row0, ROWS_PER_SUB)])

    return sc_kernel(x_aug, src2, dst2)


_BR = 2000


def _tc_body(s_ref, x_ref, w0_ref, w1_ref, lw_ref, b_ref, o_ref):
    s0 = s_ref[0]
    s1 = s_ref[1]
    dn = (((1,), (0,)), ((), ()))
    hp = lax.Precision.HIGHEST
    d0 = jnp.maximum(s0[:, D:D + 1], 1.0)
    d1 = jnp.maximum(s1[:, D:D + 1], 1.0)
    acc = lax.dot_general(s0[:, :D], w0_ref[...], dn, precision=hp) / d0
    acc = acc + lax.dot_general(s1[:, :D], w1_ref[...], dn, precision=hp) / d1
    acc = acc + lax.dot_general(x_ref[...], lw_ref[...], dn, precision=hp)
    o_ref[...] = jnp.tanh(acc + b_ref[...])


def _tc_combine(S, x, W0, W1, loop_w, h_bias_row):
    return pl.pallas_call(
        _tc_body,
        grid=(N // _BR,),
        in_specs=[
            pl.BlockSpec((R, _BR, DA), lambda i: (0, i, 0)),
            pl.BlockSpec((_BR, D), lambda i: (i, 0)),
            pl.BlockSpec((D, D), lambda i: (0, 0)),
            pl.BlockSpec((D, D), lambda i: (0, 0)),
            pl.BlockSpec((D, D), lambda i: (0, 0)),
            pl.BlockSpec((1, D), lambda i: (0, 0)),
        ],
        out_specs=pl.BlockSpec((_BR, D), lambda i: (i, 0)),
        out_shape=jax.ShapeDtypeStruct((N, D), jnp.float32),
    )(S, x, W0, W1, loop_w, h_bias_row)


def kernel(x, W, loop_weight, h_bias, edge_index_rel0, edge_index_rel1):
    x_aug = jnp.concatenate([x, jnp.ones((N, L), jnp.float32)], axis=1)
    src2 = jnp.stack([edge_index_rel0[0], edge_index_rel1[0]])
    dst2 = jnp.stack([edge_index_rel0[1], edge_index_rel1[1]])
    S = _sc_segment_sum(x_aug, src2, dst2)
    return _tc_combine(S, x, W[0], W[1], loop_weight, h_bias.reshape(1, D))


# speedup vs baseline: 5.1793x; 5.1793x over previous
import functools

import jax
import jax.numpy as jnp
from jax import lax
from jax.experimental import pallas as pl
from jax.experimental.pallas import tpu as pltpu
from jax.experimental.pallas import tpu_sc as plsc

N = 10000
D = 128
E = 160000
R = 2
L = 16
NS = 16
N_PAD = 10240
ROWS_PER_SUB = N_PAD // NS
E_PER_SUB = E // NS
CHUNK = 80


def _sc_segment_sum(x, src2, dst2):
    mesh = plsc.VectorSubcoreMesh(core_axis_name="c", subcore_axis_name="s")

    @functools.partial(
        pl.kernel,
        out_type=(jax.ShapeDtypeStruct((R, N_PAD, D), jnp.float32),
                  jax.ShapeDtypeStruct((R, N_PAD), jnp.float32)),
        mesh=mesh,
        scratch_types=[
            pltpu.VMEM((CHUNK,), jnp.int32),
            pltpu.VMEM((CHUNK,), jnp.int32),
            pltpu.VMEM((CHUNK, D), jnp.float32),
            pltpu.VMEM((CHUNK,), jnp.float32),
            pltpu.VMEM((CHUNK,), jnp.float32),
            pltpu.VMEM_SHARED((N_PAD, D), jnp.float32),
            pltpu.VMEM_SHARED((N_PAD,), jnp.float32),
        ],
    )
    def sc_kernel(x_hbm, src_hbm, dst_hbm, out_hbm, deg_hbm,
                  src_v, dst_v, rows_v, ones_v, zeros_v, acc_sh, deg_sh):
        c = lax.axis_index("c")
        s = lax.axis_index("s")

        one = jnp.full((L,), 1.0, jnp.float32)
        zero = jnp.zeros((L,), jnp.float32)

        @pl.loop(0, CHUNK, step=L)
        def _(i):
            ones_v[pl.ds(i, L)] = one
            zeros_v[pl.ds(i, L)] = zero

        @pl.loop(0, CHUNK)
        def _(i):
            @pl.loop(0, D, step=L)
            def _(j):
                rows_v[i, pl.ds(j, L)] = zero

        row0 = s * ROWS_PER_SUB

        @pl.loop(0, ROWS_PER_SUB, step=CHUNK)
        def _(r0):
            pltpu.sync_copy(rows_v, acc_sh.at[pl.ds(row0 + r0, CHUNK)])
            pltpu.sync_copy(zeros_v, deg_sh.at[pl.ds(row0 + r0, CHUNK)])

        plsc.subcore_barrier()

        ebase = c * E + s * E_PER_SUB

        @pl.loop(0, E_PER_SUB, step=CHUNK)
        def _(e0):
            pltpu.sync_copy(src_hbm.at[pl.ds(ebase + e0, CHUNK)], src_v)
            pltpu.sync_copy(dst_hbm.at[pl.ds(ebase + e0, CHUNK)], dst_v)
            pltpu.sync_copy(x_hbm.at[src_v], rows_v)
            pltpu.sync_copy(rows_v, acc_sh.at[dst_v], add=True)
            pltpu.sync_copy(ones_v, deg_sh.at[dst_v], add=True)

        plsc.subcore_barrier()

        pltpu.sync_copy(acc_sh.at[pl.ds(row0, ROWS_PER_SUB)],
                        out_hbm.at[c, pl.ds(row0, ROWS_PER_SUB)])
        pltpu.sync_copy(deg_sh.at[pl.ds(row0, ROWS_PER_SUB)],
                        deg_hbm.at[c, pl.ds(row0, ROWS_PER_SUB)])

    return sc_kernel(x, src2, dst2)


_BR = 1024


def _tc_body(s_ref, deg_ref, x_ref, w0_ref, w1_ref, lw_ref, b_ref, o_ref):
    dn = (((1,), (0,)), ((), ()))
    hp = lax.Precision.HIGHEST
    d0 = jnp.maximum(deg_ref[0], 1.0)
    d1 = jnp.maximum(deg_ref[1], 1.0)
    acc = lax.dot_general(s_ref[0], w0_ref[...], dn, precision=hp) / d0
    acc = acc + lax.dot_general(s_ref[1], w1_ref[...], dn, precision=hp) / d1
    acc = acc + lax.dot_general(x_ref[...], lw_ref[...], dn, precision=hp)
    o_ref[...] = jnp.tanh(acc + b_ref[...])


def _tc_combine(S, deg3, x_pad, W0, W1, loop_w, h_bias_row):
    return pl.pallas_call(
        _tc_body,
        grid=(N_PAD // _BR,),
        in_specs=[
            pl.BlockSpec((R, _BR, D), lambda i: (0, i, 0)),
            pl.BlockSpec((R, _BR, 1), lambda i: (0, i, 0)),
            pl.BlockSpec((_BR, D), lambda i: (i, 0)),
            pl.BlockSpec((D, D), lambda i: (0, 0)),
            pl.BlockSpec((D, D), lambda i: (0, 0)),
            pl.BlockSpec((D, D), lambda i: (0, 0)),
            pl.BlockSpec((1, D), lambda i: (0, 0)),
        ],
        out_specs=pl.BlockSpec((_BR, D), lambda i: (i, 0)),
        out_shape=jax.ShapeDtypeStruct((N_PAD, D), jnp.float32),
    )(S, deg3, x_pad, W0, W1, loop_w, h_bias_row)


def kernel(x, W, loop_weight, h_bias, edge_index_rel0, edge_index_rel1):
    src2 = jnp.concatenate([edge_index_rel0[0], edge_index_rel1[0]])
    dst2 = jnp.concatenate([edge_index_rel0[1], edge_index_rel1[1]])
    S, deg = _sc_segment_sum(x, src2, dst2)
    deg3 = deg.reshape(R, N_PAD, 1)
    x_pad = jnp.pad(x, ((0, N_PAD - N), (0, 0)))
    h_pad = _tc_combine(S, deg3, x_pad, W[0], W[1], loop_weight,
                        h_bias.reshape(1, D))
    return h_pad[:N]


# speedup vs baseline: 10.6972x; 2.0654x over previous
import functools

import jax
import jax.numpy as jnp
from jax import lax
from jax.experimental import pallas as pl
from jax.experimental.pallas import tpu as pltpu
from jax.experimental.pallas import tpu_sc as plsc

N = 10000
D = 128
E = 160000
R = 2
L = 16
NS = 16
N_PAD = 10240
ROWS_PER_SUB = N_PAD // NS
E_PER_SUB = E // NS
CHUNK = 80
NCHUNK = E_PER_SUB // CHUNK


def _sc_segment_sum(x, src_flat, dst_flat):
    mesh = plsc.VectorSubcoreMesh(core_axis_name="c", subcore_axis_name="s")

    @functools.partial(
        pl.kernel,
        out_type=(jax.ShapeDtypeStruct((R, N_PAD, D), jnp.float32),
                  jax.ShapeDtypeStruct((R, N_PAD), jnp.float32)),
        mesh=mesh,
        scratch_types=[
            pltpu.VMEM((E_PER_SUB,), jnp.int32),
            pltpu.VMEM((CHUNK,), jnp.int32),
            pltpu.VMEM((CHUNK,), jnp.int32),
            pltpu.VMEM((CHUNK, D), jnp.float32),
            pltpu.VMEM((CHUNK, D), jnp.float32),
            pltpu.VMEM((CHUNK,), jnp.float32),
            pltpu.VMEM((CHUNK,), jnp.float32),
            pltpu.VMEM_SHARED((N_PAD, D), jnp.float32),
            pltpu.VMEM_SHARED((N_PAD,), jnp.float32),
            pltpu.SemaphoreType.DMA,
            pltpu.SemaphoreType.DMA,
            pltpu.SemaphoreType.DMA,
            pltpu.SemaphoreType.DMA,
        ],
    )
    def sc_kernel(x_hbm, src_hbm, dst_hbm, out_hbm, deg_hbm,
                  src_v, dst0, dst1, rows0, rows1, ones_v, zeros_v,
                  acc_sh, deg_sh, sem0, sem1, sem2, sem3):
        c = lax.axis_index("c")
        s = lax.axis_index("s")
        ebase = c * E + s * E_PER_SUB

        one = jnp.full((L,), 1.0, jnp.float32)
        zero = jnp.zeros((L,), jnp.float32)

        @pl.loop(0, CHUNK, step=L)
        def _(i):
            ones_v[pl.ds(i, L)] = one
            zeros_v[pl.ds(i, L)] = zero

        @pl.loop(0, 64)
        def _(i):
            @pl.loop(0, D, step=L)
            def _(j):
                rows0[i, pl.ds(j, L)] = zero

        pltpu.async_copy(src_hbm.at[pl.ds(ebase, E_PER_SUB)], src_v, sem0).wait()

        row0 = s * ROWS_PER_SUB

        @pl.loop(0, ROWS_PER_SUB, step=64)
        def _(r0):
            pltpu.sync_copy(rows0.at[pl.ds(0, 64)],
                            acc_sh.at[pl.ds(row0 + r0, 64)])
            pltpu.sync_copy(zeros_v.at[pl.ds(0, 64)],
                            deg_sh.at[pl.ds(row0 + r0, 64)])

        plsc.subcore_barrier()

        def gather(k, buf, sem):
            return pltpu.make_async_copy(
                x_hbm.at[src_v.at[pl.ds(k * CHUNK, CHUNK)]], buf, sem)

        def dstcp(k, buf, sem):
            return pltpu.make_async_copy(
                dst_hbm.at[pl.ds(ebase + k * CHUNK, CHUNK)], buf, sem)

        def scatter(buf, dbuf):
            pltpu.sync_copy(buf, acc_sh.at[dbuf], add=True)
            pltpu.sync_copy(ones_v, deg_sh.at[dbuf], add=True)

        dstcp(0, dst0, sem2).start()
        dstcp(1, dst1, sem3).start()
        gather(0, rows0, sem0).start()

        @pl.loop(0, NCHUNK - 1, step=2)
        def _(a):
            b = a + 1
            gather(b, rows1, sem1).start()
            gather(a, rows0, sem0).wait()
            dstcp(a, dst0, sem2).wait()
            scatter(rows0, dst0)
            gather(a + 2, rows0, sem0).start()
            dstcp(a + 2, dst0, sem2).start()
            gather(b, rows1, sem1).wait()
            dstcp(b, dst1, sem3).wait()
            scatter(rows1, dst1)

            @pl.when(b + 2 < NCHUNK)
            def _():
                dstcp(b + 2, dst1, sem3).start()

        gather(NCHUNK - 1, rows0, sem0).wait()
        dstcp(NCHUNK - 1, dst0, sem2).wait()
        scatter(rows0, dst0)

        plsc.subcore_barrier()

        pltpu.sync_copy(acc_sh.at[pl.ds(row0, ROWS_PER_SUB)],
                        out_hbm.at[c, pl.ds(row0, ROWS_PER_SUB)])
        pltpu.sync_copy(deg_sh.at[pl.ds(row0, ROWS_PER_SUB)],
                        deg_hbm.at[c, pl.ds(row0, ROWS_PER_SUB)])

    return sc_kernel(x, src_flat, dst_flat)


_BR = 1024


def _tc_body(s_ref, deg_ref, x_ref, w0_ref, w1_ref, lw_ref, b_ref, o_ref):
    dn = (((1,), (0,)), ((), ()))
    hp = lax.Precision.HIGHEST
    d0 = jnp.maximum(deg_ref[0], 1.0)
    d1 = jnp.maximum(deg_ref[1], 1.0)
    acc = lax.dot_general(s_ref[0], w0_ref[...], dn, precision=hp) / d0
    acc = acc + lax.dot_general(s_ref[1], w1_ref[...], dn, precision=hp) / d1
    acc = acc + lax.dot_general(x_ref[...], lw_ref[...], dn, precision=hp)
    o_ref[...] = jnp.tanh(acc + b_ref[...])


def _tc_combine(S, deg3, x_pad, W0, W1, loop_w, h_bias_row):
    return pl.pallas_call(
        _tc_body,
        grid=(N_PAD // _BR,),
        in_specs=[
            pl.BlockSpec((R, _BR, D), lambda i: (0, i, 0)),
            pl.BlockSpec((R, _BR, 1), lambda i: (0, i, 0)),
            pl.BlockSpec((_BR, D), lambda i: (i, 0)),
            pl.BlockSpec((D, D), lambda i: (0, 0)),
            pl.BlockSpec((D, D), lambda i: (0, 0)),
            pl.BlockSpec((D, D), lambda i: (0, 0)),
            pl.BlockSpec((1, D), lambda i: (0, 0)),
        ],
        out_specs=pl.BlockSpec((_BR, D), lambda i: (i, 0)),
        out_shape=jax.ShapeDtypeStruct((N_PAD, D), jnp.float32),
    )(S, deg3, x_pad, W0, W1, loop_w, h_bias_row)


def kernel(x, W, loop_weight, h_bias, edge_index_rel0, edge_index_rel1):
    src_flat = jnp.concatenate([edge_index_rel0[0], edge_index_rel1[0]])
    dst_flat = jnp.concatenate([edge_index_rel0[1], edge_index_rel1[1]])
    S, deg = _sc_segment_sum(x, src_flat, dst_flat)
    deg3 = deg.reshape(R, N_PAD, 1)
    x_pad = jnp.pad(x, ((0, N_PAD - N), (0, 0)))
    h_pad = _tc_combine(S, deg3, x_pad, W[0], W[1], loop_weight,
                        h_bias.reshape(1, D))
    return h_pad[:N]


# speedup vs baseline: 11.7685x; 1.1001x over previous
import functools

import jax
import jax.numpy as jnp
from jax import lax
from jax.experimental import pallas as pl
from jax.experimental.pallas import tpu as pltpu
from jax.experimental.pallas import tpu_sc as plsc

N = 10000
D = 128
E = 160000
R = 2
L = 16
NS = 16
N_PAD = 10240
ROWS_PER_SUB = N_PAD // NS
E_PER_SUB = E // NS
CHUNK = 80
NCHUNK = 126
E_SUB_PAD = NCHUNK * CHUNK


def _sc_segment_sum(x, src_flat, dst_flat):
    mesh = plsc.VectorSubcoreMesh(core_axis_name="c", subcore_axis_name="s")

    @functools.partial(
        pl.kernel,
        out_type=(jax.ShapeDtypeStruct((R, N_PAD, D), jnp.float32),
                  jax.ShapeDtypeStruct((R, N_PAD), jnp.float32)),
        mesh=mesh,
        scratch_types=[
            pltpu.VMEM((E_SUB_PAD,), jnp.int32),
            pltpu.VMEM((CHUNK,), jnp.int32),
            pltpu.VMEM((CHUNK,), jnp.int32),
            pltpu.VMEM((CHUNK,), jnp.int32),
            pltpu.VMEM((CHUNK, D), jnp.float32),
            pltpu.VMEM((CHUNK, D), jnp.float32),
            pltpu.VMEM((CHUNK, D), jnp.float32),
            pltpu.VMEM((CHUNK,), jnp.float32),
            pltpu.VMEM((CHUNK,), jnp.float32),
            pltpu.VMEM_SHARED((N_PAD, D), jnp.float32),
            pltpu.VMEM_SHARED((N_PAD,), jnp.float32),
            pltpu.SemaphoreType.DMA,
            pltpu.SemaphoreType.DMA,
            pltpu.SemaphoreType.DMA,
            pltpu.SemaphoreType.DMA,
            pltpu.SemaphoreType.DMA,
            pltpu.SemaphoreType.DMA,
        ],
    )
    def sc_kernel(x_hbm, src_hbm, dst_hbm, out_hbm, deg_hbm,
                  src_v, dst0, dst1, dst2, rows0, rows1, rows2,
                  ones_v, zeros_v, acc_sh, deg_sh,
                  sg0, sg1, sg2, sd0, sd1, sd2):
        c = lax.axis_index("c")
        s = lax.axis_index("s")
        ebase = (c * NS + s) * E_SUB_PAD

        one = jnp.full((L,), 1.0, jnp.float32)
        zero = jnp.zeros((L,), jnp.float32)

        @pl.loop(0, CHUNK, step=L)
        def _(i):
            ones_v[pl.ds(i, L)] = one
            zeros_v[pl.ds(i, L)] = zero

        @pl.loop(0, 64)
        def _(i):
            @pl.loop(0, D, step=L)
            def _(j):
                rows0[i, pl.ds(j, L)] = zero

        pltpu.async_copy(src_hbm.at[pl.ds(ebase, E_SUB_PAD)], src_v, sg0).wait()

        row0 = s * ROWS_PER_SUB

        @pl.loop(0, ROWS_PER_SUB, step=64)
        def _(r0):
            pltpu.sync_copy(rows0.at[pl.ds(0, 64)],
                            acc_sh.at[pl.ds(row0 + r0, 64)])
            pltpu.sync_copy(zeros_v.at[pl.ds(0, 64)],
                            deg_sh.at[pl.ds(row0 + r0, 64)])

        plsc.subcore_barrier()

        def gather(k, buf, sem):
            return pltpu.make_async_copy(
                x_hbm.at[src_v.at[pl.ds(k * CHUNK, CHUNK)]], buf, sem)

        def dstcp(k, buf, sem):
            return pltpu.make_async_copy(
                dst_hbm.at[pl.ds(ebase + k * CHUNK, CHUNK)], buf, sem)

        def consume(k, buf, dbuf, sg, sd):
            gather(k, buf, sg).wait()
            dstcp(k, dbuf, sd).wait()
            pltpu.sync_copy(buf, acc_sh.at[dbuf], add=True)
            pltpu.sync_copy(ones_v, deg_sh.at[dbuf], add=True)

        def prefetch(k, buf, dbuf, sg, sd):
            @pl.when(k < NCHUNK)
            def _():
                dstcp(k, dbuf, sd).start()
                gather(k, buf, sg).start()

        dstcp(0, dst0, sd0).start()
        gather(0, rows0, sg0).start()
        dstcp(1, dst1, sd1).start()
        gather(1, rows1, sg1).start()

        @pl.loop(0, NCHUNK, step=3)
        def _(a):
            prefetch(a + 2, rows2, dst2, sg2, sd2)
            consume(a, rows0, dst0, sg0, sd0)
            prefetch(a + 3, rows0, dst0, sg0, sd0)
            consume(a + 1, rows1, dst1, sg1, sd1)
            prefetch(a + 4, rows1, dst1, sg1, sd1)
            consume(a + 2, rows2, dst2, sg2, sd2)

        plsc.subcore_barrier()

        pltpu.sync_copy(acc_sh.at[pl.ds(row0, ROWS_PER_SUB)],
                        out_hbm.at[c, pl.ds(row0, ROWS_PER_SUB)])
        pltpu.sync_copy(deg_sh.at[pl.ds(row0, ROWS_PER_SUB)],
                        deg_hbm.at[c, pl.ds(row0, ROWS_PER_SUB)])

    return sc_kernel(x, src_flat, dst_flat)


_BR = 1024


def _tc_body(s_ref, deg_ref, x_ref, w0_ref, w1_ref, lw_ref, b_ref, o_ref):
    dn = (((1,), (0,)), ((), ()))
    hp = lax.Precision.HIGHEST
    d0 = jnp.maximum(deg_ref[0], 1.0)
    d1 = jnp.maximum(deg_ref[1], 1.0)
    acc = lax.dot_general(s_ref[0], w0_ref[...], dn, precision=hp) / d0
    acc = acc + lax.dot_general(s_ref[1], w1_ref[...], dn, precision=hp) / d1
    acc = acc + lax.dot_general(x_ref[...], lw_ref[...], dn, precision=hp)
    o_ref[...] = jnp.tanh(acc + b_ref[...])


def _tc_combine(S, deg3, x_pad, W0, W1, loop_w, h_bias_row):
    return pl.pallas_call(
        _tc_body,
        grid=(N_PAD // _BR,),
        in_specs=[
            pl.BlockSpec((R, _BR, D), lambda i: (0, i, 0)),
            pl.BlockSpec((R, _BR, 1), lambda i: (0, i, 0)),
            pl.BlockSpec((_BR, D), lambda i: (i, 0)),
            pl.BlockSpec((D, D), lambda i: (0, 0)),
            pl.BlockSpec((D, D), lambda i: (0, 0)),
            pl.BlockSpec((D, D), lambda i: (0, 0)),
            pl.BlockSpec((1, D), lambda i: (0, 0)),
        ],
        out_specs=pl.BlockSpec((_BR, D), lambda i: (i, 0)),
        out_shape=jax.ShapeDtypeStruct((N_PAD, D), jnp.float32),
    )(S, deg3, x_pad, W0, W1, loop_w, h_bias_row)


def _pad_edges(idx, fake):
    return jnp.concatenate(
        [idx.reshape(NS, E_PER_SUB), fake], axis=1).reshape(-1)


def kernel(x, W, loop_weight, h_bias, edge_index_rel0, edge_index_rel1):
    fake_src = jnp.broadcast_to(
        (jnp.arange(CHUNK, dtype=jnp.int32) * 125) % N, (NS, CHUNK))
    fake_dst = jnp.broadcast_to(
        N + (jnp.arange(CHUNK, dtype=jnp.int32) * 3) % (N_PAD - N),
        (NS, CHUNK))
    src_flat = jnp.concatenate(
        [_pad_edges(edge_index_rel0[0], fake_src),
         _pad_edges(edge_index_rel1[0], fake_src)])
    dst_flat = jnp.concatenate(
        [_pad_edges(edge_index_rel0[1], fake_dst),
         _pad_edges(edge_index_rel1[1], fake_dst)])
    S, deg = _sc_segment_sum(x, src_flat, dst_flat)
    deg3 = deg.reshape(R, N_PAD, 1)
    x_pad = jnp.pad(x, ((0, N_PAD - N), (0, 0)))
    h_pad = _tc_combine(S, deg3, x_pad, W[0], W[1], loop_weight,
                        h_bias.reshape(1, D))
    return h_pad[:N]
